# baseline (speedup 1.0000x reference)
import functools

import jax
import jax.numpy as jnp
from jax import lax
from jax.experimental import pallas as pl
from jax.experimental.pallas import tpu as pltpu
from jax.experimental.pallas import tpu_sc as plsc

_L = 16


_CPR = 32


def _mean_body(x_ref, s_ref, o_ref):
    xb = x_ref[...].astype(jnp.bfloat16)
    acc = jnp.dot(xb, s_ref[...], preferred_element_type=jnp.float32)
    o_ref[...] = acc


def _mean_tc(x2, hw):
    R, E = x2.shape
    seg = (jnp.arange(E)[:, None] // hw
           == jnp.arange(_CPR)[None, :]).astype(jnp.bfloat16)
    RB = 512
    grid = R // RB
    return pl.pallas_call(
        _mean_body,
        grid=(grid,),
        in_specs=[
            pl.BlockSpec((RB, E), lambda i: (i, 0)),
            pl.BlockSpec((E, _CPR), lambda i: (0, 0)),
        ],
        out_specs=pl.BlockSpec((RB, _CPR), lambda i: (i, 0)),
        out_shape=jax.ShapeDtypeStruct((R, _CPR), jnp.float32),
    )(x2, seg)


def _mlp_body(y_ref, w1t_ref, b1_ref, w2t_ref, b2_ref, o_ref, *, scale):
    y = y_ref[...] * scale
    h = jnp.dot(y, w1t_ref[...], preferred_element_type=jnp.float32)
    h = h + b1_ref[...]
    h = jnp.where(h >= 0, h, 0.01 * h)
    h = jnp.dot(h, w2t_ref[...], preferred_element_type=jnp.float32)
    h = h + b2_ref[...]
    o_ref[...] = jax.nn.sigmoid(h)


def _mlp_tc(ysum, W1, b1, W2, b2, scale):
    B, C = ysum.shape
    HID = W1.shape[0]
    return pl.pallas_call(
        functools.partial(_mlp_body, scale=scale),
        out_shape=jax.ShapeDtypeStruct((B, C), jnp.float32),
    )(ysum, W1.T, b1.reshape(1, HID), W2.T, b2.reshape(1, C))


def _gate_tc(x3, W1, b1, W2, b2):
    B, C, HW = x3.shape
    sums = _mean_tc(x3.reshape(B * C // _CPR, _CPR * HW), HW)
    return _mlp_tc(sums.reshape(B, C), W1, b1, W2, b2, 1.0 / HW)


def _make_sc_select(B, C, K):
    NW = 32
    SPW = B // NW
    NCH = C // _L

    mesh = plsc.VectorSubcoreMesh(
        core_axis_name="c", subcore_axis_name="s", num_cores=2, num_subcores=16
    )

    def body(yi_hbm, sel_hbm, exc_hbm, yv, selv, excv, hist):
        wid = lax.axis_index("s") * 2 + lax.axis_index("c")
        pltpu.sync_copy(yi_hbm.at[pl.ds(wid * SPW * C, SPW * C)], yv)

        iota = lax.iota(jnp.int32, _L)
        zeros = jnp.zeros((_L,), jnp.int32)

        def per_sample(s, _):
            yoff = s * C

            kk = jnp.full((_L,), K, jnp.int32)
            prefix = zeros
            for p in range(4):
                sh = 24 - 8 * p

                def zero_hist(j, _c):
                    hist[pl.ds(j * _L, _L)] = zeros
                    return 0
                lax.fori_loop(0, _L, zero_hist, 0, unroll=True)

                def count_chunk(c, _c, sh=sh, first=(p == 0)):
                    u = yv[pl.ds(yoff + c * _L, _L)]
                    q = lax.shift_right_logical(u, jnp.full((_L,), sh, jnp.int32))
                    b = q & 255
                    if first:
                        cnt, last = plsc.scan_count(b)
                    else:
                        m = lax.shift_right_logical(q, jnp.full((_L,), 8, jnp.int32)) == prefix
                        cnt, last = plsc.scan_count(b, mask=m)
                    plsc.addupdate_scatter(hist, [b], cnt, mask=last)
                    return 0
                lax.fori_loop(0, NCH, count_chunk, 0, unroll=4)

                G = zeros
                for l in range(_L):
                    G = G + plsc.load_gather(hist, [iota * _L + l])
                RG = lax.rev(plsc.cumsum(lax.rev(G, (0,))), (0,))
                geh = RG >= kk
                Dh = plsc.all_reduce_population_count(geh) - 1
                gt_h = jnp.sum(jnp.where(iota > Dh, G, 0))
                Lv = plsc.load_gather(hist, [Dh * _L + iota])
                RL = lax.rev(plsc.cumsum(lax.rev(Lv, (0,))), (0,)) + gt_h
                gel = RL >= kk
                Dl = plsc.all_reduce_population_count(gel) - 1
                cnt_gt = gt_h + jnp.sum(jnp.where(iota > Dl, Lv, 0))
                kk = kk - cnt_gt
                prefix = lax.shift_left(prefix, 8) | (lax.shift_left(Dh, 4) | Dl)

            thr = prefix
            slots = kk

            def compact(c, carry):
                gbase, tie = carry
                u = yv[pl.ds(yoff + c * _L, _L)]
                gt = u > thr
                eq = u == thr
                gti = jnp.where(gt, 1, 0)
                eqi = jnp.where(eq, 1, 0)
                gpref = plsc.cumsum(gti)
                epref = plsc.cumsum(eqi)
                gx = gpref - gti
                ex = epref - eqi
                eq_rank = ex + tie
                sel = gt | (eq & (eq_rank < slots))
                acc_before = jnp.minimum(ex + tie, slots)
                pos = s * K + gbase + gx + acc_before
                idxv = iota + c * _L
                plsc.store_scatter(selv, [pos], idxv, mask=sel)
                epos = s * K + (c * _L + iota) - (gbase + gx + acc_before)
                plsc.store_scatter(excv, [epos], idxv, mask=jnp.logical_not(sel))
                return gbase + jnp.sum(gti), tie + jnp.sum(eqi)

            lax.fori_loop(0, NCH, compact, (jnp.int32(0), jnp.int32(0)),
                          unroll=2)
            return 0

        lax.fori_loop(0, SPW, per_sample, 0)

        pltpu.sync_copy(selv, sel_hbm.at[pl.ds(wid * SPW * K, SPW * K)])
        pltpu.sync_copy(excv, exc_hbm.at[pl.ds(wid * SPW * K, SPW * K)])

    return pl.kernel(
        body,
        out_type=(
            jax.ShapeDtypeStruct((B * K,), jnp.int32),
            jax.ShapeDtypeStruct((B * K,), jnp.int32),
        ),
        mesh=mesh,
        compiler_params=pltpu.CompilerParams(needs_layout_passes=False),
        scratch_types=[
            pltpu.VMEM((SPW * C,), jnp.int32),
            pltpu.VMEM((SPW * K,), jnp.int32),
            pltpu.VMEM((SPW * K,), jnp.int32),
            pltpu.VMEM((_L * _L,), jnp.int32),
        ],
    )


def kernel(x, W1, b1, W2, b2):
    B, C, H, W = x.shape
    K = 384
    y = _gate_tc(x.reshape(B, C, H * W), W1, b1, W2, b2)
    yi = lax.bitcast_convert_type(y, jnp.int32).reshape(B * C)
    sel, exc = _make_sc_select(B, C, K)(yi)
    return (
        y.reshape(B, C, 1, 1),
        sel.reshape(B, K, 1, 1),
        exc.reshape(B, K, 1, 1),
    )


# speedup vs baseline: 11.9229x; 11.9229x over previous
import functools

import jax
import jax.numpy as jnp
from jax import lax
from jax.experimental import pallas as pl
from jax.experimental.pallas import tpu as pltpu
from jax.experimental.pallas import tpu_sc as plsc

_L = 16


def _gate_body(x_ref, w1t_ref, b1_ref, w2t_ref, b2_ref, y_ref, *, scale):
    y = jnp.sum(x_ref[...], axis=0) * scale
    h = jnp.dot(y, w1t_ref[...], preferred_element_type=jnp.float32)
    h = h + b1_ref[...]
    h = jnp.where(h >= 0, h, 0.01 * h)
    h = jnp.dot(h, w2t_ref[...], preferred_element_type=jnp.float32)
    h = h + b2_ref[...]
    y_ref[...] = jax.nn.sigmoid(h)


def _gate_tc(xt, W1, b1, W2, b2):
    HW, B, C = xt.shape
    HID = W1.shape[0]
    BB = 16
    grid = B // BB
    return pl.pallas_call(
        functools.partial(_gate_body, scale=1.0 / HW),
        grid=(grid,),
        in_specs=[
            pl.BlockSpec((HW, BB, C), lambda i: (0, i, 0)),
            pl.BlockSpec((C, HID), lambda i: (0, 0)),
            pl.BlockSpec((1, HID), lambda i: (0, 0)),
            pl.BlockSpec((HID, C), lambda i: (0, 0)),
            pl.BlockSpec((1, C), lambda i: (0, 0)),
        ],
        out_specs=pl.BlockSpec((BB, C), lambda i: (i, 0)),
        out_shape=jax.ShapeDtypeStruct((B, C), jnp.float32),
    )(xt, W1.T, b1.reshape(1, HID), W2.T, b2.reshape(1, C))


def _make_sc_select(B, C, K):
    NW = 32
    SPW = B // NW
    NCH = C // _L

    mesh = plsc.VectorSubcoreMesh(
        core_axis_name="c", subcore_axis_name="s", num_cores=2, num_subcores=16
    )

    def body(yi_hbm, sel_hbm, exc_hbm, yv, selv, excv, hist):
        wid = lax.axis_index("s") * 2 + lax.axis_index("c")
        pltpu.sync_copy(yi_hbm.at[pl.ds(wid * SPW * C, SPW * C)], yv)

        iota = lax.iota(jnp.int32, _L)
        zeros = jnp.zeros((_L,), jnp.int32)

        def per_sample(s, _):
            yoff = s * C

            kk = jnp.full((_L,), K, jnp.int32)
            prefix = zeros
            for p in range(4):
                sh = 24 - 8 * p

                def zero_hist(j, _c):
                    hist[pl.ds(j * _L, _L)] = zeros
                    return 0
                lax.fori_loop(0, _L, zero_hist, 0, unroll=True)

                def count_chunk(c, _c, sh=sh, first=(p == 0)):
                    u = yv[pl.ds(yoff + c * _L, _L)]
                    q = lax.shift_right_logical(u, jnp.full((_L,), sh, jnp.int32))
                    b = q & 255
                    if first:
                        cnt, last = plsc.scan_count(b)
                    else:
                        m = lax.shift_right_logical(q, jnp.full((_L,), 8, jnp.int32)) == prefix
                        cnt, last = plsc.scan_count(b, mask=m)
                    plsc.addupdate_scatter(hist, [b], cnt, mask=last)
                    return 0
                lax.fori_loop(0, NCH, count_chunk, 0, unroll=4)

                G = zeros
                for l in range(_L):
                    G = G + plsc.load_gather(hist, [iota * _L + l])
                RG = lax.rev(plsc.cumsum(lax.rev(G, (0,))), (0,))
                geh = RG >= kk
                Dh = plsc.all_reduce_population_count(geh) - 1
                gt_h = jnp.sum(jnp.where(iota > Dh, G, 0))
                Lv = plsc.load_gather(hist, [Dh * _L + iota])
                RL = lax.rev(plsc.cumsum(lax.rev(Lv, (0,))), (0,)) + gt_h
                gel = RL >= kk
                Dl = plsc.all_reduce_population_count(gel) - 1
                cnt_gt = gt_h + jnp.sum(jnp.where(iota > Dl, Lv, 0))
                kk = kk - cnt_gt
                prefix = lax.shift_left(prefix, 8) | (lax.shift_left(Dh, 4) | Dl)

            thr = prefix
            slots = kk

            def compact(c, carry):
                gbase, tie = carry
                u = yv[pl.ds(yoff + c * _L, _L)]
                gt = u > thr
                eq = u == thr
                gti = jnp.where(gt, 1, 0)
                eqi = jnp.where(eq, 1, 0)
                gpref = plsc.cumsum(gti)
                epref = plsc.cumsum(eqi)
                gx = gpref - gti
                ex = epref - eqi
                eq_rank = ex + tie
                sel = gt | (eq & (eq_rank < slots))
                acc_before = jnp.minimum(ex + tie, slots)
                pos = s * K + gbase + gx + acc_before
                idxv = iota + c * _L
                plsc.store_scatter(selv, [pos], idxv, mask=sel)
                epos = s * K + (c * _L + iota) - (gbase + gx + acc_before)
                plsc.store_scatter(excv, [epos], idxv, mask=jnp.logical_not(sel))
                return gbase + jnp.sum(gti), tie + jnp.sum(eqi)

            lax.fori_loop(0, NCH, compact, (jnp.int32(0), jnp.int32(0)),
                          unroll=2)
            return 0

        lax.fori_loop(0, SPW, per_sample, 0)

        pltpu.sync_copy(selv, sel_hbm.at[pl.ds(wid * SPW * K, SPW * K)])
        pltpu.sync_copy(excv, exc_hbm.at[pl.ds(wid * SPW * K, SPW * K)])

    return pl.kernel(
        body,
        out_type=(
            jax.ShapeDtypeStruct((B * K,), jnp.int32),
            jax.ShapeDtypeStruct((B * K,), jnp.int32),
        ),
        mesh=mesh,
        compiler_params=pltpu.CompilerParams(needs_layout_passes=False),
        scratch_types=[
            pltpu.VMEM((SPW * C,), jnp.int32),
            pltpu.VMEM((SPW * K,), jnp.int32),
            pltpu.VMEM((SPW * K,), jnp.int32),
            pltpu.VMEM((_L * _L,), jnp.int32),
        ],
    )


def kernel(x, W1, b1, W2, b2):
    B, C, H, W = x.shape
    K = 384
    xt = x.transpose(2, 3, 0, 1).reshape(H * W, B, C)
    y = _gate_tc(xt, W1, b1, W2, b2)
    yi = lax.bitcast_convert_type(y, jnp.int32).reshape(B * C)
    sel, exc = _make_sc_select(B, C, K)(yi)
    return (
        y.reshape(B, C, 1, 1),
        sel.reshape(B, K, 1, 1),
        exc.reshape(B, K, 1, 1),
    )


# speedup vs baseline: 13.0711x; 1.0963x over previous
import functools

import jax
import jax.numpy as jnp
from jax import lax
from jax.experimental import pallas as pl
from jax.experimental.pallas import tpu as pltpu
from jax.experimental.pallas import tpu_sc as plsc

_L = 16


def _gate_body(x_ref, w1t_ref, b1_ref, w2t_ref, b2_ref, y_ref, *, scale):
    y = jnp.sum(x_ref[...], axis=0) * scale
    h = jnp.dot(y, w1t_ref[...], preferred_element_type=jnp.float32)
    h = h + b1_ref[...]
    h = jnp.where(h >= 0, h, 0.01 * h)
    h = jnp.dot(h, w2t_ref[...], preferred_element_type=jnp.float32)
    h = h + b2_ref[...]
    y_ref[...] = jax.nn.sigmoid(h)


def _gate_tc(xt, W1, b1, W2, b2, b0, bn):
    HW, B, C = xt.shape
    HID = W1.shape[0]
    BB = 16
    grid = bn // BB
    blk0 = b0 // BB
    return pl.pallas_call(
        functools.partial(_gate_body, scale=1.0 / HW),
        grid=(grid,),
        in_specs=[
            pl.BlockSpec((HW, BB, C), lambda i: (0, blk0 + i, 0)),
            pl.BlockSpec((C, HID), lambda i: (0, 0)),
            pl.BlockSpec((1, HID), lambda i: (0, 0)),
            pl.BlockSpec((HID, C), lambda i: (0, 0)),
            pl.BlockSpec((1, C), lambda i: (0, 0)),
        ],
        out_specs=pl.BlockSpec((BB, C), lambda i: (i, 0)),
        out_shape=jax.ShapeDtypeStruct((bn, C), jnp.float32),
    )(xt, W1.T, b1.reshape(1, HID), W2.T, b2.reshape(1, C))


def _make_sc_select(B, C, K):
    NW = 32
    SPW = B // NW
    NCH = C // _L

    mesh = plsc.VectorSubcoreMesh(
        core_axis_name="c", subcore_axis_name="s", num_cores=2, num_subcores=16
    )

    def body(yi_hbm, sel_hbm, exc_hbm, yv, selv, excv, hist):
        wid = lax.axis_index("s") * 2 + lax.axis_index("c")
        pltpu.sync_copy(yi_hbm.at[pl.ds(wid * SPW * C, SPW * C)], yv)

        iota = lax.iota(jnp.int32, _L)
        zeros = jnp.zeros((_L,), jnp.int32)

        def per_sample(s, _):
            yoff = s * C

            kk = jnp.full((_L,), K, jnp.int32)
            prefix = zeros
            for p in range(4):
                sh = 24 - 8 * p

                def zero_hist(j, _c):
                    hist[pl.ds(j * _L, _L)] = zeros
                    return 0
                lax.fori_loop(0, _L, zero_hist, 0, unroll=True)

                def count_chunk(c, _c, sh=sh, first=(p == 0)):
                    u = yv[pl.ds(yoff + c * _L, _L)]
                    q = lax.shift_right_logical(u, jnp.full((_L,), sh, jnp.int32))
                    b = q & 255
                    if first:
                        cnt, last = plsc.scan_count(b)
                    else:
                        m = lax.shift_right_logical(q, jnp.full((_L,), 8, jnp.int32)) == prefix
                        cnt, last = plsc.scan_count(b, mask=m)
                    plsc.addupdate_scatter(hist, [b], cnt, mask=last)
                    return 0
                lax.fori_loop(0, NCH, count_chunk, 0, unroll=4)

                G = zeros
                for l in range(_L):
                    G = G + plsc.load_gather(hist, [iota * _L + l])
                RG = lax.rev(plsc.cumsum(lax.rev(G, (0,))), (0,))
                geh = RG >= kk
                Dh = plsc.all_reduce_population_count(geh) - 1
                gt_h = jnp.sum(jnp.where(iota > Dh, G, 0))
                Lv = plsc.load_gather(hist, [Dh * _L + iota])
                RL = lax.rev(plsc.cumsum(lax.rev(Lv, (0,))), (0,)) + gt_h
                gel = RL >= kk
                Dl = plsc.all_reduce_population_count(gel) - 1
                cnt_gt = gt_h + jnp.sum(jnp.where(iota > Dl, Lv, 0))
                kk = kk - cnt_gt
                prefix = lax.shift_left(prefix, 8) | (lax.shift_left(Dh, 4) | Dl)

            thr = prefix
            slots = kk

            def compact(c, carry):
                gbase, tie = carry
                u = yv[pl.ds(yoff + c * _L, _L)]
                gt = u > thr
                eq = u == thr
                gti = jnp.where(gt, 1, 0)
                eqi = jnp.where(eq, 1, 0)
                gpref = plsc.cumsum(gti)
                epref = plsc.cumsum(eqi)
                gx = gpref - gti
                ex = epref - eqi
                eq_rank = ex + tie
                sel = gt | (eq & (eq_rank < slots))
                acc_before = jnp.minimum(ex + tie, slots)
                pos = s * K + gbase + gx + acc_before
                idxv = iota + c * _L
                plsc.store_scatter(selv, [pos], idxv, mask=sel)
                epos = s * K + (c * _L + iota) - (gbase + gx + acc_before)
                plsc.store_scatter(excv, [epos], idxv, mask=jnp.logical_not(sel))
                return gbase + jnp.sum(gti), tie + jnp.sum(eqi)

            lax.fori_loop(0, NCH, compact, (jnp.int32(0), jnp.int32(0)),
                          unroll=2)
            return 0

        lax.fori_loop(0, SPW, per_sample, 0)

        pltpu.sync_copy(selv, sel_hbm.at[pl.ds(wid * SPW * K, SPW * K)])
        pltpu.sync_copy(excv, exc_hbm.at[pl.ds(wid * SPW * K, SPW * K)])

    return pl.kernel(
        body,
        out_type=(
            jax.ShapeDtypeStruct((B * K,), jnp.int32),
            jax.ShapeDtypeStruct((B * K,), jnp.int32),
        ),
        mesh=mesh,
        compiler_params=pltpu.CompilerParams(needs_layout_passes=False),
        scratch_types=[
            pltpu.VMEM((SPW * C,), jnp.int32),
            pltpu.VMEM((SPW * K,), jnp.int32),
            pltpu.VMEM((SPW * K,), jnp.int32),
            pltpu.VMEM((_L * _L,), jnp.int32),
        ],
    )


def kernel(x, W1, b1, W2, b2):
    B, C, H, W = x.shape
    K = 384
    xt = x.transpose(2, 3, 0, 1).reshape(H * W, B, C)
    NS = 2
    Bh = B // NS
    sc_call = _make_sc_select(Bh, C, K)
    ys, sels, excs = [], [], []
    for i in range(NS):
        y = _gate_tc(xt, W1, b1, W2, b2, i * Bh, Bh)
        yi = lax.bitcast_convert_type(y, jnp.int32).reshape(Bh * C)
        sel, exc = sc_call(yi)
        ys.append(y)
        sels.append(sel.reshape(Bh, K, 1, 1))
        excs.append(exc.reshape(Bh, K, 1, 1))
    y = jnp.concatenate(ys, axis=0)
    return (
        y.reshape(B, C, 1, 1),
        jnp.concatenate(sels, axis=0),
        jnp.concatenate(excs, axis=0),
    )


# speedup vs baseline: 14.0114x; 1.0719x over previous
import functools

import jax
import jax.numpy as jnp
from jax import lax
from jax.experimental import pallas as pl
from jax.experimental.pallas import tpu as pltpu
from jax.experimental.pallas import tpu_sc as plsc

_L = 16


def _gate_body(x_ref, w1t_ref, b1_ref, w2t_ref, b2_ref, y_ref, *, scale):
    y = jnp.sum(x_ref[...], axis=0) * scale
    h = jnp.dot(y, w1t_ref[...], preferred_element_type=jnp.float32)
    h = h + b1_ref[...]
    h = jnp.where(h >= 0, h, 0.01 * h)
    h = jnp.dot(h, w2t_ref[...], preferred_element_type=jnp.float32)
    h = h + b2_ref[...]
    y_ref[...] = jax.nn.sigmoid(h)


def _gate_tc(xt, W1, b1, W2, b2, b0, bn):
    HW, B, C = xt.shape
    HID = W1.shape[0]
    BB = 16
    grid = bn // BB
    blk0 = b0 // BB
    return pl.pallas_call(
        functools.partial(_gate_body, scale=1.0 / HW),
        grid=(grid,),
        in_specs=[
            pl.BlockSpec((HW, BB, C), lambda i: (0, blk0 + i, 0)),
            pl.BlockSpec((C, HID), lambda i: (0, 0)),
            pl.BlockSpec((1, HID), lambda i: (0, 0)),
            pl.BlockSpec((HID, C), lambda i: (0, 0)),
            pl.BlockSpec((1, C), lambda i: (0, 0)),
        ],
        out_specs=pl.BlockSpec((BB, C), lambda i: (i, 0)),
        out_shape=jax.ShapeDtypeStruct((bn, C), jnp.float32),
    )(xt, W1.T, b1.reshape(1, HID), W2.T, b2.reshape(1, C))


def _make_sc_select(B, C, K):
    NW = 32
    SPW = B // NW
    NCH = C // _L

    mesh = plsc.VectorSubcoreMesh(
        core_axis_name="c", subcore_axis_name="s", num_cores=2, num_subcores=16
    )

    def body(yi_hbm, sel_hbm, exc_hbm, yv, selv, excv, hist):
        wid = lax.axis_index("s") * 2 + lax.axis_index("c")
        pltpu.sync_copy(yi_hbm.at[pl.ds(wid * SPW * C, SPW * C)], yv)

        iota = lax.iota(jnp.int32, _L)
        zeros = jnp.zeros((_L,), jnp.int32)

        def per_sample(s, _):
            yoff = s * C

            kk = jnp.full((_L,), K, jnp.int32)
            prefix = zeros
            for p in range(4):
                sh = 24 - 8 * p

                def zero_hist(j, _c):
                    hist[pl.ds(j * _L, _L)] = zeros
                    return 0
                lax.fori_loop(0, _L, zero_hist, 0, unroll=True)

                def count_chunk(c, _c, sh=sh, first=(p == 0)):
                    u = yv[pl.ds(yoff + c * _L, _L)]
                    q = lax.shift_right_logical(u, jnp.full((_L,), sh, jnp.int32))
                    b = q & 255
                    if first:
                        cnt, last = plsc.scan_count(b)
                    else:
                        m = lax.shift_right_logical(q, jnp.full((_L,), 8, jnp.int32)) == prefix
                        cnt, last = plsc.scan_count(b, mask=m)
                    plsc.addupdate_scatter(hist, [b], cnt, mask=last)
                    return 0
                lax.fori_loop(0, NCH, count_chunk, 0, unroll=4)

                G = zeros
                for l in range(_L):
                    G = G + plsc.load_gather(hist, [iota * _L + l])
                RG = lax.rev(plsc.cumsum(lax.rev(G, (0,))), (0,))
                geh = RG >= kk
                Dh = plsc.all_reduce_population_count(geh) - 1
                gt_h = jnp.sum(jnp.where(iota > Dh, G, 0))
                Lv = plsc.load_gather(hist, [Dh * _L + iota])
                RL = lax.rev(plsc.cumsum(lax.rev(Lv, (0,))), (0,)) + gt_h
                gel = RL >= kk
                Dl = plsc.all_reduce_population_count(gel) - 1
                cnt_gt = gt_h + jnp.sum(jnp.where(iota > Dl, Lv, 0))
                kk = kk - cnt_gt
                prefix = lax.shift_left(prefix, 8) | (lax.shift_left(Dh, 4) | Dl)

            thr = prefix
            slots = kk

            def compact(c, carry):
                gbase, tie = carry
                u = yv[pl.ds(yoff + c * _L, _L)]
                gt = u > thr
                eq = u == thr
                gti = jnp.where(gt, 1, 0)
                eqi = jnp.where(eq, 1, 0)
                gpref = plsc.cumsum(gti)
                epref = plsc.cumsum(eqi)
                gx = gpref - gti
                ex = epref - eqi
                eq_rank = ex + tie
                sel = gt | (eq & (eq_rank < slots))
                acc_before = jnp.minimum(ex + tie, slots)
                pos = s * K + gbase + gx + acc_before
                idxv = iota + c * _L
                plsc.store_scatter(selv, [pos], idxv, mask=sel)
                epos = s * K + (c * _L + iota) - (gbase + gx + acc_before)
                plsc.store_scatter(excv, [epos], idxv, mask=jnp.logical_not(sel))
                return (gbase + plsc.all_reduce_population_count(gt),
                        tie + plsc.all_reduce_population_count(eq))

            lax.fori_loop(0, NCH, compact, (zeros, zeros), unroll=2)
            return 0

        lax.fori_loop(0, SPW, per_sample, 0)

        pltpu.sync_copy(selv, sel_hbm.at[pl.ds(wid * SPW * K, SPW * K)])
        pltpu.sync_copy(excv, exc_hbm.at[pl.ds(wid * SPW * K, SPW * K)])

    return pl.kernel(
        body,
        out_type=(
            jax.ShapeDtypeStruct((B * K,), jnp.int32),
            jax.ShapeDtypeStruct((B * K,), jnp.int32),
        ),
        mesh=mesh,
        compiler_params=pltpu.CompilerParams(needs_layout_passes=False),
        scratch_types=[
            pltpu.VMEM((SPW * C,), jnp.int32),
            pltpu.VMEM((SPW * K,), jnp.int32),
            pltpu.VMEM((SPW * K,), jnp.int32),
            pltpu.VMEM((_L * _L,), jnp.int32),
        ],
    )


def kernel(x, W1, b1, W2, b2):
    B, C, H, W = x.shape
    K = 384
    xt = x.transpose(2, 3, 0, 1).reshape(H * W, B, C)
    NS = 2
    Bh = B // NS
    sc_call = _make_sc_select(Bh, C, K)
    ys, sels, excs = [], [], []
    for i in range(NS):
        y = _gate_tc(xt, W1, b1, W2, b2, i * Bh, Bh)
        yi = lax.bitcast_convert_type(y, jnp.int32).reshape(Bh * C)
        sel, exc = sc_call(yi)
        ys.append(y)
        sels.append(sel.reshape(Bh, K, 1, 1))
        excs.append(exc.reshape(Bh, K, 1, 1))
    y = jnp.concatenate(ys, axis=0)
    return (
        y.reshape(B, C, 1, 1),
        jnp.concatenate(sels, axis=0),
        jnp.concatenate(excs, axis=0),
    )
